# baseline (speedup 1.0000x reference)
import functools

import jax
import jax.numpy as jnp
from jax import lax
from jax.experimental import pallas as pl
from jax.experimental.pallas import tpu as pltpu
from jax.experimental.pallas import tpu_sc as plsc

EMBED = 64
SCALE = 8.0

_info = plsc.get_sparse_core_info()
_NC, _NS, _L = _info.num_cores, _info.num_subcores, _info.num_lanes
_NW = _NC * _NS


@functools.partial(jax.jit, static_argnums=(2, 3))
def _lookup(x_flat, tbl, n_rows, row_len):
    rows_per_w = n_rows // _NW
    pad_len = ((row_len + _L - 1) // _L) * _L
    mesh = plsc.VectorSubcoreMesh(core_axis_name="c", subcore_axis_name="s")

    @functools.partial(
        pl.kernel,
        out_type=jax.ShapeDtypeStruct((n_rows, row_len, EMBED), jnp.float32),
        mesh=mesh,
        scratch_types=[
            pltpu.VMEM((pad_len,), jnp.int32),
            pltpu.VMEM((pad_len, 2 * EMBED), jnp.float32),
            pltpu.VMEM((pad_len, EMBED), jnp.float32),
            pltpu.SemaphoreType.DMA,
        ],
        compiler_params=pltpu.CompilerParams(
            use_tc_tiling_on_sc=True, needs_layout_passes=False
        ),
    )
    def k(x_hbm, tbl_hbm, out_hbm, idx_v, rows_v, out_v, sem):
        wid = lax.axis_index("s") * _NC + lax.axis_index("c")
        base = wid * rows_per_w
        lane = lax.iota(jnp.int32, _L)

        def row_body(g, carry):
            r = base + g
            pltpu.sync_copy(x_hbm.at[pl.ds(r * row_len, row_len)],
                            idx_v.at[pl.ds(0, row_len)])
            tail = idx_v[pl.ds(pad_len - _L, _L)]
            tail = jnp.where(lane < (row_len - (pad_len - _L)), tail, 0)
            idx_v[pl.ds(pad_len - _L, _L)] = tail

            pltpu.async_copy(tbl_hbm.at[idx_v], rows_v, sem).wait()

            def scale_j(j, c2):
                for c in range(EMBED // _L):
                    sl = pl.ds(c * _L, _L)
                    out_v[j, sl] = rows_v[j, sl] * SCALE
                return c2

            lax.fori_loop(0, row_len, scale_j, 0)
            pltpu.sync_copy(out_v.at[pl.ds(0, row_len)], out_hbm.at[r])
            return carry

        lax.fori_loop(0, rows_per_w, row_body, 0)

    return k(x_flat, tbl)


def kernel(x, embedding_table):
    n_rows, row_len = x.shape
    x_flat = x.reshape(-1).astype(jnp.int32)
    tbl = jnp.pad(embedding_table, ((0, 0), (0, EMBED)))
    return _lookup(x_flat, tbl, n_rows, row_len)
